# Initial kernel scaffold; baseline (speedup 1.0000x reference)
import jax
import jax.numpy as jnp
from jax.experimental import pallas as pl


def kernel(x, edge_index, emb_table, W_in, b_in, ln1_g, ln1_b, W_msg, b_msg, ln2_g, ln2_b, W_out, b_out):
    raise NotImplementedError("write your pallas kernel here")


# speedup vs baseline: 5.0434x; 5.0434x over previous
import functools

import jax
import jax.numpy as jnp
from jax import lax
from jax.experimental import pallas as pl
from jax.experimental.pallas import tpu as pltpu
from jax.experimental.pallas import tpu_sc as plsc

N_NODES = 10000
CH = 128
EMB_CH = 32
NC = 2
NS = 16
NW = NC * NS
CHUNK = 128
AGG_ROWS = 10112
ROWS_PER_SUB = AGG_ROWS // NS
DUMMY_DST = 10008
HIST_ROWS = 80


def _silu_ln(z, g, b):
    z = z * (1.0 / (1.0 + jnp.exp(-z)))
    mu = jnp.mean(z, axis=-1, keepdims=True)
    var = jnp.mean((z - mu) * (z - mu), axis=-1, keepdims=True)
    return (z - mu) * jax.lax.rsqrt(var + 1e-5) * g + b


def _tc1_body(x_ref, emb_ref, wemb_ref, wx_ref, bin_ref, g_ref, b_ref, h_ref):
    x = x_ref[...]
    blk = x.shape[0]
    cols = lax.broadcasted_iota(jnp.int32, (blk, CH), 1)
    opc = x[:, 0:1].astype(jnp.int32)
    onehot = (cols == opc).astype(jnp.float32)
    w_emb = jnp.dot(emb_ref[...], wemb_ref[...], preferred_element_type=jnp.float32)
    xz = jnp.where(cols == 0, 0.0, x)
    z = (jnp.dot(onehot, w_emb, preferred_element_type=jnp.float32)
         + jnp.dot(xz, wx_ref[...], preferred_element_type=jnp.float32)
         + bin_ref[...])
    h_ref[...] = _silu_ln(z, g_ref[...], b_ref[...])


def _tc2_body(h_ref, agg2_ref, deg_ref, wmh_ref, wma_ref, bm_ref, g_ref,
              b_ref, wo_ref, bo_ref, out_ref):
    h = h_ref[...]
    a = agg2_ref[0] + agg2_ref[1]
    a = a / jnp.maximum(deg_ref[...], 1.0)
    z = (jnp.dot(h, wmh_ref[...], preferred_element_type=jnp.float32)
         + jnp.dot(a, wma_ref[...], preferred_element_type=jnp.float32)
         + bm_ref[...])
    z = _silu_ln(z, g_ref[...], b_ref[...])
    out_ref[...] = jnp.sum(z * wo_ref[...], axis=-1, keepdims=True) + bo_ref[...]


def _sc_agg_body(h_hbm, srcs_hbm, dsts_hbm, zeros_hbm, agg_out,
                 src_all, dst_all, rows_v, agg_sh, sem):
    c = lax.axis_index("c")
    s = lax.axis_index("s")
    wid = s * NC + c
    nchunks = AGG_ROWS // CHUNK

    pltpu.sync_copy(srcs_hbm.at[wid], src_all)
    pltpu.sync_copy(dsts_hbm.at[wid], dst_all)
    pltpu.sync_copy(zeros_hbm, rows_v)

    @pl.loop(s, nchunks, step=NS)
    def _zero(j):
        pltpu.sync_copy(rows_v, agg_sh.at[pl.ds(j * CHUNK, CHUNK)])

    plsc.subcore_barrier()

    @pl.loop(0, src_all.shape[0])
    def _edges(j):
        pltpu.async_copy(h_hbm.at[src_all.at[j]], rows_v, sem).wait()
        pltpu.sync_copy(rows_v, agg_sh.at[dst_all.at[j]], add=True)

    plsc.subcore_barrier()

    @pl.loop(s, nchunks, step=NS)
    def _drain(j):
        rows = pl.ds(j * CHUNK, CHUNK)
        pltpu.sync_copy(agg_sh.at[rows], rows_v)
        pltpu.sync_copy(rows_v, agg_out.at[c, rows])


def _sc_deg_body(dsts_hbm, zeros_hbm, ones_hbm, deg_out,
                 dst_all, zeros_v, ones_v, deg_sh):
    c = lax.axis_index("c")
    s = lax.axis_index("s")
    wid = s * NC + c
    nchunks = AGG_ROWS // CHUNK

    pltpu.sync_copy(dsts_hbm.at[wid], dst_all)
    pltpu.sync_copy(zeros_hbm, zeros_v)
    pltpu.sync_copy(ones_hbm, ones_v)

    @pl.loop(s, nchunks, step=NS)
    def _zero(j):
        pltpu.sync_copy(zeros_v, deg_sh.at[pl.ds(j * CHUNK, CHUNK)])

    plsc.subcore_barrier()

    @pl.loop(0, dst_all.shape[0])
    def _edges(j):
        pltpu.sync_copy(ones_v, deg_sh.at[dst_all.at[j]], add=True)

    plsc.subcore_barrier()

    @pl.loop(s, nchunks, step=NS)
    def _drain(j):
        rows = pl.ds(j * CHUNK, CHUNK)
        pltpu.sync_copy(deg_sh.at[rows], zeros_v)
        pltpu.sync_copy(zeros_v, deg_out.at[c, rows])


def kernel(x, edge_index, emb_table, W_in, b_in, ln1_g, ln1_b, W_msg, b_msg,
           ln2_g, ln2_b, W_out, b_out):
    f32 = jnp.float32
    n = x.shape[0]
    e = edge_index.shape[1]

    wemb = W_in[:EMB_CH]
    wx = jnp.concatenate([jnp.zeros((1, CH), f32), W_in[EMB_CH:]], axis=0)
    blk1 = 1000
    h = pl.pallas_call(
        _tc1_body,
        grid=(n // blk1,),
        in_specs=[
            pl.BlockSpec((blk1, CH), lambda i: (i, 0)),
            pl.BlockSpec((CH, EMB_CH), lambda i: (0, 0)),
            pl.BlockSpec((EMB_CH, CH), lambda i: (0, 0)),
            pl.BlockSpec((CH, CH), lambda i: (0, 0)),
            pl.BlockSpec((1, CH), lambda i: (0, 0)),
            pl.BlockSpec((1, CH), lambda i: (0, 0)),
            pl.BlockSpec((1, CH), lambda i: (0, 0)),
        ],
        out_specs=pl.BlockSpec((blk1, CH), lambda i: (i, 0)),
        out_shape=jax.ShapeDtypeStruct((n, CH), f32),
    )(x, emb_table, wemb, wx, b_in.reshape(1, CH), ln1_g.reshape(1, CH),
      ln1_b.reshape(1, CH))

    epw = ((e // NW + CHUNK - 1) // CHUNK) * CHUNK
    nsteps = epw // CHUNK
    e_pad = epw * NW
    src = edge_index[0].astype(jnp.int32)
    dst = edge_index[1].astype(jnp.int32)
    srcs = jnp.concatenate([src, jnp.zeros((e_pad - e,), jnp.int32)])
    dsts = jnp.concatenate(
        [dst, jnp.full((e_pad - e,), DUMMY_DST, jnp.int32)])
    srcs = srcs.reshape(NW, nsteps, CHUNK)
    dsts = dsts.reshape(NW, nsteps, CHUNK)
    zeros = jnp.zeros((CHUNK, CH), f32)
    ones = jnp.ones((CHUNK, CH), f32)

    mesh = plsc.VectorSubcoreMesh(core_axis_name="c", subcore_axis_name="s")
    deg2 = pl.kernel(
        _sc_deg_body,
        out_type=jax.ShapeDtypeStruct((NC, AGG_ROWS, CH), f32),
        mesh=mesh,
        scratch_types=[
            pltpu.VMEM((nsteps, CHUNK), jnp.int32),
            pltpu.VMEM((CHUNK, CH), f32),
            pltpu.VMEM((CHUNK, CH), f32),
            pltpu.VMEM_SHARED((AGG_ROWS, CH), f32),
        ],
    )(dsts, zeros, ones)
    agg2 = pl.kernel(
        _sc_agg_body,
        out_type=jax.ShapeDtypeStruct((NC, AGG_ROWS, CH), f32),
        mesh=mesh,
        scratch_types=[
            pltpu.VMEM((nsteps, CHUNK), jnp.int32),
            pltpu.VMEM((nsteps, CHUNK), jnp.int32),
            pltpu.VMEM((CHUNK, CH), f32),
            pltpu.VMEM_SHARED((AGG_ROWS, CH), f32),
            pltpu.SemaphoreType.DMA,
        ],
    )(h, srcs, dsts, zeros)
    deg_col = (deg2[0, :, 0] + deg2[1, :, 0])[:n].reshape(n, 1)

    wmh = W_msg[:CH]
    wma = W_msg[CH:]
    blk2 = 1000
    out = pl.pallas_call(
        _tc2_body,
        grid=(n // blk2,),
        in_specs=[
            pl.BlockSpec((blk2, CH), lambda i: (i, 0)),
            pl.BlockSpec((NC, blk2, CH), lambda i: (0, i, 0)),
            pl.BlockSpec((blk2, 1), lambda i: (i, 0)),
            pl.BlockSpec((CH, CH), lambda i: (0, 0)),
            pl.BlockSpec((CH, CH), lambda i: (0, 0)),
            pl.BlockSpec((1, CH), lambda i: (0, 0)),
            pl.BlockSpec((1, CH), lambda i: (0, 0)),
            pl.BlockSpec((1, CH), lambda i: (0, 0)),
            pl.BlockSpec((1, CH), lambda i: (0, 0)),
            pl.BlockSpec((1, 1), lambda i: (0, 0)),
        ],
        out_specs=pl.BlockSpec((blk2, 1), lambda i: (i, 0)),
        out_shape=jax.ShapeDtypeStruct((n, 1), f32),
    )(h, agg2, deg_col, wmh, wma, b_msg.reshape(1, CH), ln2_g.reshape(1, CH),
      ln2_b.reshape(1, CH), W_out.reshape(1, CH), b_out.reshape(1, 1))
    return out[:, 0]
